# Initial kernel scaffold; baseline (speedup 1.0000x reference)
import jax
import jax.numpy as jnp
from jax.experimental import pallas as pl


def kernel(x, edge_index, batch, ginfo, W1, b1, Ws1, bs1, W2, b2, Ws2, bs2, L1w, L1b, L2w, L2b, L3w, L3b):
    raise NotImplementedError("write your pallas kernel here")


# speedup vs baseline: 1.1489x; 1.1489x over previous
import functools
import jax
import jax.numpy as jnp
from jax.experimental import pallas as pl
from jax.experimental.pallas import tpu as pltpu

N = 10000
E = 320000
B = 64
H = 128
NP = 10240


def _sortable_f32(x):
    u = jax.lax.bitcast_convert_type(x, jnp.int32)
    v = jnp.where(u < 0, ~u, u | jnp.int32(-2147483648))
    return jax.lax.bitcast_convert_type(v, jnp.uint32)


def _topk_mask(keys, batch, k):
    L = len(keys)
    prefix = [jnp.zeros((B,), jnp.uint32) for _ in range(L)]
    kf = k.astype(jnp.float32)

    def ge_lex(cand):
        gt = jnp.zeros((N,), jnp.bool_)
        eq = jnp.ones((N,), jnp.bool_)
        for l in range(L):
            c = cand[l][batch]
            gt = gt | (eq & (keys[l] > c))
            eq = eq & (keys[l] == c)
        return gt | eq

    for l in range(L):
        for bit in range(31, -1, -1):
            cand = list(prefix)
            cand[l] = prefix[l] | jnp.uint32(1 << bit)
            m = ge_lex(cand)
            cnt = jax.ops.segment_sum(m.astype(jnp.float32), batch, num_segments=B)
            prefix[l] = jnp.where(cnt >= kf, cand[l], prefix[l])
    return ge_lex(prefix)


def _head_body(xg_ref, l1w_ref, l1b_ref, l2w_ref, l2b_ref, l3w_ref, l3b_ref, out_ref):
    xg = xg_ref[...]
    a = jnp.maximum(jnp.dot(xg, l1w_ref[...], preferred_element_type=jnp.float32)
                    + l1b_ref[...][None, :], 0.0)
    a = jnp.maximum(jnp.dot(a, l2w_ref[...], preferred_element_type=jnp.float32)
                    + l2b_ref[...][None, :], 0.0)
    z = jnp.dot(a, l3w_ref[...], preferred_element_type=jnp.float32) + l3b_ref[...][None, :]
    zmax = jnp.max(z, axis=1, keepdims=True)
    ez = jnp.exp(z - zmax)
    lse = jnp.log(jnp.sum(ez, axis=1, keepdims=True)) + zmax
    out_ref[...] = z - lse


def _mlp_head(xg, L1w, L1b, L2w, L2b, L3w, L3b):
    F = xg.shape[1]
    FP = 384
    xgp = jnp.pad(xg, ((0, 0), (0, FP - F)))
    l1wp = jnp.pad(L1w, ((0, FP - F), (0, 0)))
    return pl.pallas_call(
        _head_body,
        out_shape=jax.ShapeDtypeStruct((B, 32), jnp.float32),
    )(xgp, l1wp, L1b, L2w, L2b, L3w, L3b)


def kernel(x, edge_index, batch, ginfo, W1, b1, Ws1, bs1, W2, b2, Ws2, bs2,
           L1w, L1b, L2w, L2b, L3w, L3b):
    src, dst = edge_index[0], edge_index[1]
    x0 = x[:, 0]
    ones = jnp.ones((E,), jnp.float32)
    indeg = jax.ops.segment_sum(ones, dst, num_segments=N)
    deg0 = indeg + 1.0
    dinv0 = deg0 ** -0.5
    u = dinv0 * x0
    t = jax.ops.segment_sum(u[src], dst, num_segments=N)
    a1 = dinv0 * (t + dinv0 * x0)
    h = jax.nn.relu(a1[:, None] * W1[0][None, :] + b1[None, :])
    s = h @ Ws1[:, 0]
    v = dinv0 * s
    tv = jax.ops.segment_sum(v[src], dst, num_segments=N)
    score1 = dinv0 * (tv + v) + bs1[0]

    counts0 = jax.ops.segment_sum(jnp.ones((N,), jnp.float32), batch, num_segments=B)
    k1 = jnp.ceil(0.25 * counts0).astype(jnp.int32)
    nid = jnp.arange(N, dtype=jnp.int32)
    key_s1 = _sortable_f32(score1)
    key_id = jax.lax.bitcast_convert_type(~nid, jnp.uint32)
    sel1 = _topk_mask([key_s1, key_id], batch, k1)
    sel1f = sel1.astype(jnp.float32)

    h_pool = (sel1f * jnp.tanh(score1))[:, None] * h
    neginf = jnp.float32(-jnp.inf)
    x1_max = jax.ops.segment_max(jnp.where(sel1[:, None], h_pool, neginf), batch, num_segments=B)
    x1_sum = jax.ops.segment_sum(h_pool, batch, num_segments=B) / k1.astype(jnp.float32)[:, None]

    w = jax.ops.segment_sum(sel1f[src], dst, num_segments=N)
    deg2 = 1.0 + sel1f * w
    dinv2 = deg2 ** -0.5
    p = dinv2[:, None] * h_pool
    msum = jax.ops.segment_sum(p[src] * sel1f[dst][:, None], dst, num_segments=N)
    m_total = dinv2[:, None] * msum + (dinv2 ** 2)[:, None] * h_pool
    h2 = sel1f[:, None] * jax.nn.relu(m_total @ W2 + b2[None, :])
    s2 = h2 @ Ws2[:, 0]
    v2 = dinv2 * s2
    tv2 = jax.ops.segment_sum(v2[src] * sel1f[src] * sel1f[dst], dst, num_segments=N)
    score2 = dinv2 * tv2 + dinv2 * v2 + bs2[0]

    k2 = jnp.ceil(0.25 * k1.astype(jnp.float32)).astype(jnp.int32)
    key_s2 = _sortable_f32(score2)
    Z = jnp.uint32(0)
    sel2 = _topk_mask([jnp.where(sel1, key_s2, Z), jnp.where(sel1, key_s1, Z),
                       jnp.where(sel1, key_id, Z)], batch, k2)
    sel2f = sel2.astype(jnp.float32)

    h_fin = (sel2f * jnp.tanh(score2))[:, None] * h2
    x2_max = jax.ops.segment_max(jnp.where(sel2[:, None], h_fin, neginf), batch, num_segments=B)
    x2_sum = jax.ops.segment_sum(h_fin, batch, num_segments=B) / k2.astype(jnp.float32)[:, None]

    xg = jnp.concatenate([x1_max + x2_max, x1_sum + x2_sum, ginfo], axis=1)
    return _mlp_head(xg, L1w, L1b, L2w, L2b, L3w, L3b)


# speedup vs baseline: 2.2342x; 1.9447x over previous
import functools
import jax
import jax.numpy as jnp
from jax.experimental import pallas as pl
from jax.experimental.pallas import tpu as pltpu

N = 10000
E = 320000
B = 64
H = 128
NP = 10240


INTMIN = -2147483648
ID_INIT = 0x7FFFC000


def _sortable_f32(x):
    u = jax.lax.bitcast_convert_type(x, jnp.int32)
    v = jnp.where(u < 0, ~u, u | jnp.int32(INTMIN))
    return v ^ jnp.int32(INTMIN)


def _radix_body(stage2, bitss, inits, batch_ref, *args):
    limb_refs = args[:len(bitss)]
    out_ref = args[len(bitss)]
    batchv = batch_ref[...]
    gid = jax.lax.broadcasted_iota(jnp.int32, (B, 1), 0)
    own = gid == batchv
    counts = jnp.sum(own.astype(jnp.float32), axis=1, keepdims=True)
    kk = jnp.ceil(0.25 * counts)
    if stage2:
        kk = jnp.ceil(0.25 * kk)
    gt = jnp.zeros((B, NP), jnp.bool_)
    eq = own
    for l, (bits, init) in enumerate(zip(bitss, inits)):
        limb = limb_refs[l][...]
        cntgt = jnp.sum(gt.astype(jnp.float32), axis=1, keepdims=True)

        def round_fn(i, prefix, limb=limb, eq=eq, cntgt=cntgt, bits=bits):
            shift = jax.lax.shift_left(jnp.int32(1), jnp.int32(bits - 1) - i)
            cand = prefix + shift
            m = eq & (limb >= cand)
            cnt = cntgt + jnp.sum(m.astype(jnp.float32), axis=1, keepdims=True)
            return jnp.where(cnt >= kk, cand, prefix)

        prefix0 = jnp.full((B, 1), init, jnp.int32)
        prefix = jax.lax.fori_loop(0, bits, round_fn, prefix0)
        gt = gt | (eq & (limb > prefix))
        eq = eq & (limb == prefix)
    sel = gt | eq
    out_ref[...] = jnp.sum(sel.astype(jnp.float32), axis=0, keepdims=True)


def _topk_mask(limbs, batch_pad, stage2):
    bitss = [32] * (len(limbs) - 1) + [14]
    inits = [INTMIN] * (len(limbs) - 1) + [ID_INIT]
    limbs2d = [jnp.pad(l, (0, NP - N)).reshape(1, NP) for l in limbs]
    out = pl.pallas_call(
        functools.partial(_radix_body, stage2, bitss, inits),
        out_shape=jax.ShapeDtypeStruct((1, NP), jnp.float32),
    )(batch_pad.reshape(1, NP), *limbs2d)
    return out[0, :N] > 0.5


def _head_body(xg_ref, l1w_ref, l1b_ref, l2w_ref, l2b_ref, l3w_ref, l3b_ref, out_ref):
    xg = xg_ref[...]
    a = jnp.maximum(jnp.dot(xg, l1w_ref[...], preferred_element_type=jnp.float32)
                    + l1b_ref[...][None, :], 0.0)
    a = jnp.maximum(jnp.dot(a, l2w_ref[...], preferred_element_type=jnp.float32)
                    + l2b_ref[...][None, :], 0.0)
    z = jnp.dot(a, l3w_ref[...], preferred_element_type=jnp.float32) + l3b_ref[...][None, :]
    zmax = jnp.max(z, axis=1, keepdims=True)
    ez = jnp.exp(z - zmax)
    lse = jnp.log(jnp.sum(ez, axis=1, keepdims=True)) + zmax
    out_ref[...] = z - lse


def _mlp_head(xg, L1w, L1b, L2w, L2b, L3w, L3b):
    F = xg.shape[1]
    FP = 384
    xgp = jnp.pad(xg, ((0, 0), (0, FP - F)))
    l1wp = jnp.pad(L1w, ((0, FP - F), (0, 0)))
    return pl.pallas_call(
        _head_body,
        out_shape=jax.ShapeDtypeStruct((B, 32), jnp.float32),
    )(xgp, l1wp, L1b, L2w, L2b, L3w, L3b)


def kernel(x, edge_index, batch, ginfo, W1, b1, Ws1, bs1, W2, b2, Ws2, bs2,
           L1w, L1b, L2w, L2b, L3w, L3b):
    src, dst = edge_index[0], edge_index[1]
    x0 = x[:, 0]
    ones = jnp.ones((E,), jnp.float32)
    indeg = jax.ops.segment_sum(ones, dst, num_segments=N)
    deg0 = indeg + 1.0
    dinv0 = deg0 ** -0.5
    u = dinv0 * x0
    t = jax.ops.segment_sum(u[src], dst, num_segments=N)
    a1 = dinv0 * (t + dinv0 * x0)
    h = jax.nn.relu(a1[:, None] * W1[0][None, :] + b1[None, :])
    s = h @ Ws1[:, 0]
    v = dinv0 * s
    tv = jax.ops.segment_sum(v[src], dst, num_segments=N)
    score1 = dinv0 * (tv + v) + bs1[0]

    counts0 = jax.ops.segment_sum(jnp.ones((N,), jnp.float32), batch, num_segments=B)
    k1 = jnp.ceil(0.25 * counts0).astype(jnp.int32)
    nid = jnp.arange(N, dtype=jnp.int32)
    batch_pad = jnp.pad(batch, (0, NP - N), constant_values=B)
    key_s1 = _sortable_f32(score1)
    key_id = (~nid) ^ jnp.int32(INTMIN)
    sel1 = _topk_mask([key_s1, key_id], batch_pad, False)
    sel1f = sel1.astype(jnp.float32)

    h_pool = (sel1f * jnp.tanh(score1))[:, None] * h
    neginf = jnp.float32(-jnp.inf)
    x1_max = jax.ops.segment_max(jnp.where(sel1[:, None], h_pool, neginf), batch, num_segments=B)
    x1_sum = jax.ops.segment_sum(h_pool, batch, num_segments=B) / k1.astype(jnp.float32)[:, None]

    w = jax.ops.segment_sum(sel1f[src], dst, num_segments=N)
    deg2 = 1.0 + sel1f * w
    dinv2 = deg2 ** -0.5
    p = dinv2[:, None] * h_pool
    msum = jax.ops.segment_sum(p[src] * sel1f[dst][:, None], dst, num_segments=N)
    m_total = dinv2[:, None] * msum + (dinv2 ** 2)[:, None] * h_pool
    h2 = sel1f[:, None] * jax.nn.relu(m_total @ W2 + b2[None, :])
    s2 = h2 @ Ws2[:, 0]
    v2 = dinv2 * s2
    tv2 = jax.ops.segment_sum(v2[src] * sel1f[src] * sel1f[dst], dst, num_segments=N)
    score2 = dinv2 * tv2 + dinv2 * v2 + bs2[0]

    k2 = jnp.ceil(0.25 * k1.astype(jnp.float32)).astype(jnp.int32)
    key_s2 = _sortable_f32(score2)
    imin = jnp.int32(INTMIN)
    sel2 = _topk_mask([jnp.where(sel1, key_s2, imin), jnp.where(sel1, key_s1, imin),
                       jnp.where(sel1, key_id, imin)], batch_pad, True)
    sel2f = sel2.astype(jnp.float32)

    h_fin = (sel2f * jnp.tanh(score2))[:, None] * h2
    x2_max = jax.ops.segment_max(jnp.where(sel2[:, None], h_fin, neginf), batch, num_segments=B)
    x2_sum = jax.ops.segment_sum(h_fin, batch, num_segments=B) / k2.astype(jnp.float32)[:, None]

    xg = jnp.concatenate([x1_max + x2_max, x1_sum + x2_sum, ginfo], axis=1)
    return _mlp_head(xg, L1w, L1b, L2w, L2b, L3w, L3b)


# speedup vs baseline: 6.4470x; 2.8856x over previous
import functools
import jax
import jax.numpy as jnp
from jax import lax
from jax.experimental import pallas as pl
from jax.experimental.pallas import tpu as pltpu
from jax.experimental.pallas import tpu_sc as plsc

N = 10000
E = 320000
B = 64
H = 128
NP = 10240
NW = 32
EP = 327680
EPW = EP // NW


def _spmv_body(src_hbm, dst_hbm, u_hbm, out_hbm, src_v, dst_v, ubuf, accum):
    c = lax.axis_index("c")
    s = lax.axis_index("s")
    wid = s * 2 + c
    base = wid * EPW
    pltpu.sync_copy(src_hbm.at[pl.ds(base, EPW)], src_v)
    pltpu.sync_copy(dst_hbm.at[pl.ds(base, EPW)], dst_v)
    pltpu.sync_copy(u_hbm, ubuf)
    zero16 = jnp.zeros((16,), jnp.float32)

    def zbody(i, carry):
        accum[pl.ds(i * 16, 16)] = zero16
        return carry

    lax.fori_loop(0, NP // 16, zbody, 0)

    def ebody(i, carry):
        sidx = src_v[pl.ds(i * 16, 16)]
        didx = dst_v[pl.ds(i * 16, 16)]
        vals = plsc.load_gather(ubuf, [sidx])
        plsc.addupdate_scatter(accum, [didx], vals)
        return carry

    lax.fori_loop(0, EPW // 16, ebody, 0)
    pltpu.sync_copy(accum, out_hbm.at[wid])


@jax.jit
def _spmv(src_pad, dst_pad, u_pad):
    f = pl.kernel(
        _spmv_body,
        out_type=jax.ShapeDtypeStruct((NW, NP), jnp.float32),
        mesh=plsc.VectorSubcoreMesh(core_axis_name="c", subcore_axis_name="s"),
        compiler_params=pltpu.CompilerParams(needs_layout_passes=False),
        scratch_types=[
            pltpu.VMEM((EPW,), jnp.int32),
            pltpu.VMEM((EPW,), jnp.int32),
            pltpu.VMEM((NP,), jnp.float32),
            pltpu.VMEM((NP,), jnp.float32),
        ],
    )
    return f(src_pad, dst_pad, u_pad)


INTMIN = -2147483648
ID_INIT = 0x7FFFC000


def _sortable_f32(x):
    u = jax.lax.bitcast_convert_type(x, jnp.int32)
    v = jnp.where(u < 0, ~u, u | jnp.int32(INTMIN))
    return v ^ jnp.int32(INTMIN)


def _radix_body(stage2, bitss, inits, batch_ref, *args):
    limb_refs = args[:len(bitss)]
    out_ref = args[len(bitss)]
    batchv = batch_ref[...]
    gid = jax.lax.broadcasted_iota(jnp.int32, (B, 1), 0)
    own = gid == batchv
    counts = jnp.sum(own.astype(jnp.float32), axis=1, keepdims=True)
    kk = jnp.ceil(0.25 * counts)
    if stage2:
        kk = jnp.ceil(0.25 * kk)
    gt = jnp.zeros((B, NP), jnp.bool_)
    eq = own
    for l, (bits, init) in enumerate(zip(bitss, inits)):
        limb = limb_refs[l][...]
        cntgt = jnp.sum(gt.astype(jnp.float32), axis=1, keepdims=True)

        def round_fn(i, prefix, limb=limb, eq=eq, cntgt=cntgt, bits=bits):
            shift = jax.lax.shift_left(jnp.int32(1), jnp.int32(bits - 1) - i)
            cand = prefix + shift
            m = eq & (limb >= cand)
            cnt = cntgt + jnp.sum(m.astype(jnp.float32), axis=1, keepdims=True)
            return jnp.where(cnt >= kk, cand, prefix)

        prefix0 = jnp.full((B, 1), init, jnp.int32)
        prefix = jax.lax.fori_loop(0, bits, round_fn, prefix0)
        gt = gt | (eq & (limb > prefix))
        eq = eq & (limb == prefix)
    sel = gt | eq
    out_ref[...] = jnp.sum(sel.astype(jnp.float32), axis=0, keepdims=True)


def _topk_mask(limbs, batch_pad, stage2):
    bitss = [32] * (len(limbs) - 1) + [14]
    inits = [INTMIN] * (len(limbs) - 1) + [ID_INIT]
    limbs2d = [jnp.pad(l, (0, NP - N)).reshape(1, NP) for l in limbs]
    out = pl.pallas_call(
        functools.partial(_radix_body, stage2, bitss, inits),
        out_shape=jax.ShapeDtypeStruct((1, NP), jnp.float32),
    )(batch_pad.reshape(1, NP), *limbs2d)
    return out[0, :N] > 0.5


def _head_body(xg_ref, l1w_ref, l1b_ref, l2w_ref, l2b_ref, l3w_ref, l3b_ref, out_ref):
    xg = xg_ref[...]
    a = jnp.maximum(jnp.dot(xg, l1w_ref[...], preferred_element_type=jnp.float32)
                    + l1b_ref[...][None, :], 0.0)
    a = jnp.maximum(jnp.dot(a, l2w_ref[...], preferred_element_type=jnp.float32)
                    + l2b_ref[...][None, :], 0.0)
    z = jnp.dot(a, l3w_ref[...], preferred_element_type=jnp.float32) + l3b_ref[...][None, :]
    zmax = jnp.max(z, axis=1, keepdims=True)
    ez = jnp.exp(z - zmax)
    lse = jnp.log(jnp.sum(ez, axis=1, keepdims=True)) + zmax
    out_ref[...] = z - lse


def _mlp_head(xg, L1w, L1b, L2w, L2b, L3w, L3b):
    F = xg.shape[1]
    FP = 384
    xgp = jnp.pad(xg, ((0, 0), (0, FP - F)))
    l1wp = jnp.pad(L1w, ((0, FP - F), (0, 0)))
    return pl.pallas_call(
        _head_body,
        out_shape=jax.ShapeDtypeStruct((B, 32), jnp.float32),
    )(xgp, l1wp, L1b, L2w, L2b, L3w, L3b)


def kernel(x, edge_index, batch, ginfo, W1, b1, Ws1, bs1, W2, b2, Ws2, bs2,
           L1w, L1b, L2w, L2b, L3w, L3b):
    src, dst = edge_index[0], edge_index[1]
    padE = jnp.full((EP - E,), N, jnp.int32)
    src_pad = jnp.concatenate([src, padE])
    dst_pad = jnp.concatenate([dst, padE])

    def padn(a):
        return jnp.pad(a, (0, NP - N))

    x0 = x[:, 0]
    indeg = jnp.sum(_spmv(src_pad, dst_pad, padn(jnp.ones((N,), jnp.float32))), axis=0)[:N]
    deg0 = indeg + 1.0
    dinv0 = deg0 ** -0.5
    u = dinv0 * x0
    t = jnp.sum(_spmv(src_pad, dst_pad, padn(u)), axis=0)[:N]
    a1 = dinv0 * (t + dinv0 * x0)
    h = jax.nn.relu(a1[:, None] * W1[0][None, :] + b1[None, :])
    s = h @ Ws1[:, 0]
    v = dinv0 * s
    tv = jnp.sum(_spmv(src_pad, dst_pad, padn(v)), axis=0)[:N]
    score1 = dinv0 * (tv + v) + bs1[0]

    counts0 = jax.ops.segment_sum(jnp.ones((N,), jnp.float32), batch, num_segments=B)
    k1 = jnp.ceil(0.25 * counts0).astype(jnp.int32)
    nid = jnp.arange(N, dtype=jnp.int32)
    batch_pad = jnp.pad(batch, (0, NP - N), constant_values=B)
    key_s1 = _sortable_f32(score1)
    key_id = (~nid) ^ jnp.int32(INTMIN)
    sel1 = _topk_mask([key_s1, key_id], batch_pad, False)
    sel1f = sel1.astype(jnp.float32)

    h_pool = (sel1f * jnp.tanh(score1))[:, None] * h
    neginf = jnp.float32(-jnp.inf)
    x1_max = jax.ops.segment_max(jnp.where(sel1[:, None], h_pool, neginf), batch, num_segments=B)
    x1_sum = jax.ops.segment_sum(h_pool, batch, num_segments=B) / k1.astype(jnp.float32)[:, None]

    w = jnp.sum(_spmv(src_pad, dst_pad, padn(sel1f)), axis=0)[:N]
    deg2 = 1.0 + sel1f * w
    dinv2 = deg2 ** -0.5
    p = dinv2[:, None] * h_pool
    msum = jax.ops.segment_sum(p[src] * sel1f[dst][:, None], dst, num_segments=N)
    m_total = dinv2[:, None] * msum + (dinv2 ** 2)[:, None] * h_pool
    h2 = sel1f[:, None] * jax.nn.relu(m_total @ W2 + b2[None, :])
    s2 = h2 @ Ws2[:, 0]
    v2 = dinv2 * s2
    tv2 = jnp.sum(_spmv(src_pad, dst_pad, padn(v2)), axis=0)[:N]
    score2 = dinv2 * tv2 + dinv2 * v2 + bs2[0]

    k2 = jnp.ceil(0.25 * k1.astype(jnp.float32)).astype(jnp.int32)
    key_s2 = _sortable_f32(score2)
    imin = jnp.int32(INTMIN)
    sel2 = _topk_mask([jnp.where(sel1, key_s2, imin), jnp.where(sel1, key_s1, imin),
                       jnp.where(sel1, key_id, imin)], batch_pad, True)
    sel2f = sel2.astype(jnp.float32)

    h_fin = (sel2f * jnp.tanh(score2))[:, None] * h2
    x2_max = jax.ops.segment_max(jnp.where(sel2[:, None], h_fin, neginf), batch, num_segments=B)
    x2_sum = jax.ops.segment_sum(h_fin, batch, num_segments=B) / k2.astype(jnp.float32)[:, None]

    xg = jnp.concatenate([x1_max + x2_max, x1_sum + x2_sum, ginfo], axis=1)
    return _mlp_head(xg, L1w, L1b, L2w, L2b, L3w, L3b)


# speedup vs baseline: 44.1106x; 6.8420x over previous
import functools
import jax
import jax.numpy as jnp
from jax import lax
from jax.experimental import pallas as pl
from jax.experimental.pallas import tpu as pltpu
from jax.experimental.pallas import tpu_sc as plsc

N = 10000
E = 320000
B = 64
H = 128
NP = 10240
NW = 32
EP = 327680
EPW = EP // NW


def _spmv_body(src_hbm, dst_hbm, u_hbm, out_hbm, src_v, dst_v, ubuf, accum):
    c = lax.axis_index("c")
    s = lax.axis_index("s")
    wid = s * 2 + c
    base = wid * EPW
    pltpu.sync_copy(src_hbm.at[pl.ds(base, EPW)], src_v)
    pltpu.sync_copy(dst_hbm.at[pl.ds(base, EPW)], dst_v)
    pltpu.sync_copy(u_hbm, ubuf)
    zero16 = jnp.zeros((16,), jnp.float32)

    def zbody(i, carry):
        accum[pl.ds(i * 16, 16)] = zero16
        return carry

    lax.fori_loop(0, NP // 16, zbody, 0)

    def ebody(i, carry):
        sidx = src_v[pl.ds(i * 16, 16)]
        didx = dst_v[pl.ds(i * 16, 16)]
        vals = plsc.load_gather(ubuf, [sidx])
        plsc.addupdate_scatter(accum, [didx], vals)
        return carry

    lax.fori_loop(0, EPW // 16, ebody, 0)
    pltpu.sync_copy(accum, out_hbm.at[wid])


@jax.jit
def _spmv(src_pad, dst_pad, u_pad):
    f = pl.kernel(
        _spmv_body,
        out_type=jax.ShapeDtypeStruct((NW, NP), jnp.float32),
        mesh=plsc.VectorSubcoreMesh(core_axis_name="c", subcore_axis_name="s"),
        compiler_params=pltpu.CompilerParams(needs_layout_passes=False),
        scratch_types=[
            pltpu.VMEM((EPW,), jnp.int32),
            pltpu.VMEM((EPW,), jnp.int32),
            pltpu.VMEM((NP,), jnp.float32),
            pltpu.VMEM((NP,), jnp.float32),
        ],
    )
    return f(src_pad, dst_pad, u_pad)


INTMIN = -2147483648
ID_INIT = 0x7FFFC000


def _sortable_f32(x):
    u = jax.lax.bitcast_convert_type(x, jnp.int32)
    v = jnp.where(u < 0, ~u, u | jnp.int32(INTMIN))
    return v ^ jnp.int32(INTMIN)


NCH = EPW // 128
ROWS_PER_SUB = NP // 16


NR = 5120
SPLIT = 5120
RPS = NR // 16
ZCH = 64


def _spmm_body(src_hbm, dst_hbm, sel_hbm, p_hbm, out_hbm,
               src_v, dst_v, selbuf, srcKL, dstKL, srcKH, dstKH, rowbuf,
               accum, gsem, ssem):
    c = lax.axis_index("c")
    s = lax.axis_index("s")
    wid = s * 2 + c
    base = wid * EPW
    pltpu.sync_copy(src_hbm.at[pl.ds(base, EPW)], src_v)
    pltpu.sync_copy(dst_hbm.at[pl.ds(base, EPW)], dst_v)
    pltpu.sync_copy(sel_hbm, selbuf)

    zero16 = jnp.zeros((16,), jnp.float32)

    def zrow(i, carry):
        rowbuf[i // 8, pl.ds((i % 8) * 16, 16)] = zero16
        return carry

    lax.fori_loop(0, (128 * H) // 16, zrow, 0)

    src_trash = jnp.full((16,), N, jnp.int32)
    dst_trash = jnp.zeros((16,), jnp.int32)

    def pfill(i, carry):
        srcKL[i // 8, pl.ds((i % 8) * 16, 16)] = src_trash
        dstKL[i // 8, pl.ds((i % 8) * 16, 16)] = dst_trash
        srcKH[i // 8, pl.ds((i % 8) * 16, 16)] = src_trash
        dstKH[i // 8, pl.ds((i % 8) * 16, 16)] = dst_trash
        return carry

    lax.fori_loop(0, EPW // 16, pfill, 0)

    def compact(i, carry):
        cntL, cntH = carry
        sidx = src_v[pl.ds(i * 16, 16)]
        didx = dst_v[pl.ds(i * 16, 16)]
        ks = plsc.load_gather(selbuf, [sidx])
        kd = plsc.load_gather(selbuf, [didx])
        keep = (ks * kd) > 0.5
        low = didx < SPLIT
        keepL = keep & low
        keepH = keep & (~low)
        kiL = keepL.astype(jnp.int32)
        kiH = keepH.astype(jnp.int32)
        posL = cntL + jax.lax.cumsum(kiL, axis=0) - kiL
        posH = cntH + jax.lax.cumsum(kiH, axis=0) - kiH
        plsc.store_scatter(srcKL, [jax.lax.shift_right_logical(posL, 7), posL & 127],
                           sidx, mask=keepL)
        plsc.store_scatter(dstKL, [jax.lax.shift_right_logical(posL, 7), posL & 127],
                           didx, mask=keepL)
        plsc.store_scatter(srcKH, [jax.lax.shift_right_logical(posH, 7), posH & 127],
                           sidx, mask=keepH)
        plsc.store_scatter(dstKH, [jax.lax.shift_right_logical(posH, 7), posH & 127],
                           didx - SPLIT, mask=keepH)
        return (cntL + jnp.sum(kiL), cntH + jnp.sum(kiH))

    cntL, cntH = lax.fori_loop(0, EPW // 16, compact, (jnp.int32(0), jnp.int32(0)))

    for half, (srcK, dstK, cnt) in enumerate(((srcKL, dstKL, cntL), (srcKH, dstKH, cntH))):
        for r in range(RPS // ZCH):
            pltpu.sync_copy(rowbuf.at[pl.ds(0, ZCH)],
                            accum.at[pl.ds(s * RPS + r * ZCH, ZCH)])
        plsc.subcore_barrier()

        def chunk(j, carry, srcK=srcK, dstK=dstK):
            pltpu.async_copy(p_hbm.at[srcK.at[j]], rowbuf, gsem).wait()
            pltpu.async_copy(rowbuf, accum.at[dstK.at[j]], ssem, add=True).wait()
            return carry

        lax.fori_loop(0, (cnt + 127) // 128, chunk, 0)
        plsc.subcore_barrier()
        pltpu.sync_copy(accum.at[pl.ds(s * RPS, RPS)],
                        out_hbm.at[half, c, pl.ds(s * RPS, RPS)])
        if half == 0:
            lax.fori_loop(0, (128 * H) // 16, zrow, 0)
            plsc.subcore_barrier()


@jax.jit
def _spmm(src_pad, dst_pad, sel_pad, p_pad):
    f = pl.kernel(
        _spmm_body,
        out_type=jax.ShapeDtypeStruct((2, 2, NR, H), jnp.float32),
        mesh=plsc.VectorSubcoreMesh(core_axis_name="c", subcore_axis_name="s"),
        compiler_params=pltpu.CompilerParams(needs_layout_passes=False),
        scratch_types=[
            pltpu.VMEM((EPW,), jnp.int32),
            pltpu.VMEM((EPW,), jnp.int32),
            pltpu.VMEM((NP,), jnp.float32),
            pltpu.VMEM((NCH, 128), jnp.int32),
            pltpu.VMEM((NCH, 128), jnp.int32),
            pltpu.VMEM((NCH, 128), jnp.int32),
            pltpu.VMEM((NCH, 128), jnp.int32),
            pltpu.VMEM((128, H), jnp.float32),
            pltpu.VMEM_SHARED((NR, H), jnp.float32),
            pltpu.SemaphoreType.DMA,
            pltpu.SemaphoreType.DMA,
        ],
    )
    return f(src_pad, dst_pad, sel_pad, p_pad)


def _radix_body(stage2, bitss, inits, batch_ref, *args):
    limb_refs = args[:len(bitss)]
    out_ref = args[len(bitss)]
    batchv = batch_ref[...]
    gid = jax.lax.broadcasted_iota(jnp.int32, (B, 1), 0)
    own = gid == batchv
    counts = jnp.sum(own.astype(jnp.float32), axis=1, keepdims=True)
    kk = jnp.ceil(0.25 * counts)
    if stage2:
        kk = jnp.ceil(0.25 * kk)
    gt = jnp.zeros((B, NP), jnp.bool_)
    eq = own
    for l, (bits, init) in enumerate(zip(bitss, inits)):
        limb = limb_refs[l][...]
        cntgt = jnp.sum(gt.astype(jnp.float32), axis=1, keepdims=True)

        def round_fn(i, prefix, limb=limb, eq=eq, cntgt=cntgt, bits=bits):
            shift = jax.lax.shift_left(jnp.int32(1), jnp.int32(bits - 1) - i)
            cand = prefix + shift
            m = eq & (limb >= cand)
            cnt = cntgt + jnp.sum(m.astype(jnp.float32), axis=1, keepdims=True)
            return jnp.where(cnt >= kk, cand, prefix)

        prefix0 = jnp.full((B, 1), init, jnp.int32)
        prefix = jax.lax.fori_loop(0, bits, round_fn, prefix0)
        gt = gt | (eq & (limb > prefix))
        eq = eq & (limb == prefix)
    sel = gt | eq
    out_ref[...] = jnp.sum(sel.astype(jnp.float32), axis=0, keepdims=True)


def _topk_mask(limbs, batch_pad, stage2):
    bitss = [32] * (len(limbs) - 1) + [14]
    inits = [INTMIN] * (len(limbs) - 1) + [ID_INIT]
    limbs2d = [jnp.pad(l, (0, NP - N)).reshape(1, NP) for l in limbs]
    out = pl.pallas_call(
        functools.partial(_radix_body, stage2, bitss, inits),
        out_shape=jax.ShapeDtypeStruct((1, NP), jnp.float32),
    )(batch_pad.reshape(1, NP), *limbs2d)
    return out[0, :N] > 0.5


def _head_body(xg_ref, l1w_ref, l1b_ref, l2w_ref, l2b_ref, l3w_ref, l3b_ref, out_ref):
    xg = xg_ref[...]
    a = jnp.maximum(jnp.dot(xg, l1w_ref[...], preferred_element_type=jnp.float32)
                    + l1b_ref[...][None, :], 0.0)
    a = jnp.maximum(jnp.dot(a, l2w_ref[...], preferred_element_type=jnp.float32)
                    + l2b_ref[...][None, :], 0.0)
    z = jnp.dot(a, l3w_ref[...], preferred_element_type=jnp.float32) + l3b_ref[...][None, :]
    zmax = jnp.max(z, axis=1, keepdims=True)
    ez = jnp.exp(z - zmax)
    lse = jnp.log(jnp.sum(ez, axis=1, keepdims=True)) + zmax
    out_ref[...] = z - lse


def _mlp_head(xg, L1w, L1b, L2w, L2b, L3w, L3b):
    F = xg.shape[1]
    FP = 384
    xgp = jnp.pad(xg, ((0, 0), (0, FP - F)))
    l1wp = jnp.pad(L1w, ((0, FP - F), (0, 0)))
    return pl.pallas_call(
        _head_body,
        out_shape=jax.ShapeDtypeStruct((B, 32), jnp.float32),
    )(xgp, l1wp, L1b, L2w, L2b, L3w, L3b)


def kernel(x, edge_index, batch, ginfo, W1, b1, Ws1, bs1, W2, b2, Ws2, bs2,
           L1w, L1b, L2w, L2b, L3w, L3b):
    src, dst = edge_index[0], edge_index[1]
    padE = jnp.full((EP - E,), N, jnp.int32)
    src_pad = jnp.concatenate([src, padE])
    dst_pad = jnp.concatenate([dst, padE])

    def padn(a):
        return jnp.pad(a, (0, NP - N))

    x0 = x[:, 0]
    indeg = jnp.sum(_spmv(src_pad, dst_pad, padn(jnp.ones((N,), jnp.float32))), axis=0)[:N]
    deg0 = indeg + 1.0
    dinv0 = deg0 ** -0.5
    u = dinv0 * x0
    t = jnp.sum(_spmv(src_pad, dst_pad, padn(u)), axis=0)[:N]
    a1 = dinv0 * (t + dinv0 * x0)
    h = jax.nn.relu(a1[:, None] * W1[0][None, :] + b1[None, :])
    s = h @ Ws1[:, 0]
    v = dinv0 * s
    tv = jnp.sum(_spmv(src_pad, dst_pad, padn(v)), axis=0)[:N]
    score1 = dinv0 * (tv + v) + bs1[0]

    counts0 = jax.ops.segment_sum(jnp.ones((N,), jnp.float32), batch, num_segments=B)
    k1 = jnp.ceil(0.25 * counts0).astype(jnp.int32)
    nid = jnp.arange(N, dtype=jnp.int32)
    batch_pad = jnp.pad(batch, (0, NP - N), constant_values=B)
    key_s1 = _sortable_f32(score1)
    key_id = (~nid) ^ jnp.int32(INTMIN)
    sel1 = _topk_mask([key_s1, key_id], batch_pad, False)
    sel1f = sel1.astype(jnp.float32)

    h_pool = (sel1f * jnp.tanh(score1))[:, None] * h
    neginf = jnp.float32(-jnp.inf)
    x1_max = jax.ops.segment_max(jnp.where(sel1[:, None], h_pool, neginf), batch, num_segments=B)
    x1_sum = jax.ops.segment_sum(h_pool, batch, num_segments=B) / k1.astype(jnp.float32)[:, None]

    w = jnp.sum(_spmv(src_pad, dst_pad, padn(sel1f)), axis=0)[:N]
    deg2 = 1.0 + sel1f * w
    dinv2 = deg2 ** -0.5
    p = dinv2[:, None] * h_pool
    p_pad = jnp.pad(p, ((0, NP - N), (0, 0)))
    sel_pad = padn(sel1f)
    mparts = _spmm(src_pad, dst_pad, sel_pad, p_pad)
    msum = jnp.concatenate([jnp.sum(mparts[0], axis=0)[:SPLIT],
                            jnp.sum(mparts[1], axis=0)[:N - SPLIT]], axis=0)
    m_total = dinv2[:, None] * msum + (dinv2 ** 2)[:, None] * h_pool
    h2 = sel1f[:, None] * jax.nn.relu(m_total @ W2 + b2[None, :])
    s2 = h2 @ Ws2[:, 0]
    v2 = dinv2 * s2
    tv2 = jnp.sum(_spmv(src_pad, dst_pad, padn(v2)), axis=0)[:N]
    score2 = dinv2 * tv2 + dinv2 * v2 + bs2[0]

    k2 = jnp.ceil(0.25 * k1.astype(jnp.float32)).astype(jnp.int32)
    key_s2 = _sortable_f32(score2)
    imin = jnp.int32(INTMIN)
    sel2 = _topk_mask([jnp.where(sel1, key_s2, imin), jnp.where(sel1, key_s1, imin),
                       jnp.where(sel1, key_id, imin)], batch_pad, True)
    sel2f = sel2.astype(jnp.float32)

    h_fin = (sel2f * jnp.tanh(score2))[:, None] * h2
    x2_max = jax.ops.segment_max(jnp.where(sel2[:, None], h_fin, neginf), batch, num_segments=B)
    x2_sum = jax.ops.segment_sum(h_fin, batch, num_segments=B) / k2.astype(jnp.float32)[:, None]

    xg = jnp.concatenate([x1_max + x2_max, x1_sum + x2_sum, ginfo], axis=1)
    return _mlp_head(xg, L1w, L1b, L2w, L2b, L3w, L3b)


# speedup vs baseline: 47.0284x; 1.0661x over previous
import functools
import jax
import jax.numpy as jnp
from jax import lax
from jax.experimental import pallas as pl
from jax.experimental.pallas import tpu as pltpu
from jax.experimental.pallas import tpu_sc as plsc

N = 10000
E = 320000
B = 64
H = 128
NP = 10240
NW = 32
EP = 327680
EPW = EP // NW


def _spmv_body(src_hbm, dst_hbm, u_hbm, out_hbm, src_v, dst_v, ubuf, accum):
    c = lax.axis_index("c")
    s = lax.axis_index("s")
    wid = s * 2 + c
    base = wid * EPW
    pltpu.sync_copy(src_hbm.at[pl.ds(base, EPW)], src_v)
    pltpu.sync_copy(dst_hbm.at[pl.ds(base, EPW)], dst_v)
    pltpu.sync_copy(u_hbm, ubuf)
    zero16 = jnp.zeros((16,), jnp.float32)

    def zbody(i, carry):
        accum[pl.ds(i * 16, 16)] = zero16
        return carry

    lax.fori_loop(0, NP // 16, zbody, 0)

    def ebody(i, carry):
        sidx = src_v[pl.ds(i * 16, 16)]
        didx = dst_v[pl.ds(i * 16, 16)]
        vals = plsc.load_gather(ubuf, [sidx])
        plsc.addupdate_scatter(accum, [didx], vals)
        return carry

    lax.fori_loop(0, EPW // 16, ebody, 0)
    pltpu.sync_copy(accum, out_hbm.at[wid])


@jax.jit
def _spmv(src_pad, dst_pad, u_pad):
    f = pl.kernel(
        _spmv_body,
        out_type=jax.ShapeDtypeStruct((NW, NP), jnp.float32),
        mesh=plsc.VectorSubcoreMesh(core_axis_name="c", subcore_axis_name="s"),
        compiler_params=pltpu.CompilerParams(needs_layout_passes=False),
        scratch_types=[
            pltpu.VMEM((EPW,), jnp.int32),
            pltpu.VMEM((EPW,), jnp.int32),
            pltpu.VMEM((NP,), jnp.float32),
            pltpu.VMEM((NP,), jnp.float32),
        ],
    )
    return f(src_pad, dst_pad, u_pad)


INTMIN = -2147483648
ID_INIT = 0x7FFFC000


def _sortable_f32(x):
    u = jax.lax.bitcast_convert_type(x, jnp.int32)
    v = jnp.where(u < 0, ~u, u | jnp.int32(INTMIN))
    return v ^ jnp.int32(INTMIN)


NCH = EPW // 128
ROWS_PER_SUB = NP // 16


NR = 5120
SPLIT = 5120
RPS = NR // 16
ZCH = 64


def _spmm_body(src_hbm, dst_hbm, sel_hbm, p_hbm, out_hbm,
               src_v, dst_v, selbuf, srcKL, dstKL, srcKH, dstKH, rowbuf,
               accum, gsem, ssem):
    c = lax.axis_index("c")
    s = lax.axis_index("s")
    wid = s * 2 + c
    base = wid * EPW
    pltpu.sync_copy(src_hbm.at[pl.ds(base, EPW)], src_v)
    pltpu.sync_copy(dst_hbm.at[pl.ds(base, EPW)], dst_v)
    pltpu.sync_copy(sel_hbm, selbuf)

    zero16 = jnp.zeros((16,), jnp.float32)

    def zrow(i, carry):
        rowbuf[i // 8, pl.ds((i % 8) * 16, 16)] = zero16
        return carry

    lax.fori_loop(0, (128 * H) // 16, zrow, 0)

    src_trash = jnp.full((16,), N, jnp.int32)
    dst_trash = jnp.zeros((16,), jnp.int32)

    def pfill(i, carry):
        srcKL[i // 8, pl.ds((i % 8) * 16, 16)] = src_trash
        dstKL[i // 8, pl.ds((i % 8) * 16, 16)] = dst_trash
        srcKH[i // 8, pl.ds((i % 8) * 16, 16)] = src_trash
        dstKH[i // 8, pl.ds((i % 8) * 16, 16)] = dst_trash
        return carry

    lax.fori_loop(0, EPW // 16, pfill, 0)

    def compact(i, carry):
        cntL, cntH = carry
        sidx = src_v[pl.ds(i * 16, 16)]
        didx = dst_v[pl.ds(i * 16, 16)]
        ks = plsc.load_gather(selbuf, [sidx])
        kd = plsc.load_gather(selbuf, [didx])
        keep = (ks * kd) > 0.5
        low = didx < SPLIT
        keepL = keep & low
        keepH = keep & (~low)
        kiL = keepL.astype(jnp.int32)
        kiH = keepH.astype(jnp.int32)
        posL = cntL + jax.lax.cumsum(kiL, axis=0) - kiL
        posH = cntH + jax.lax.cumsum(kiH, axis=0) - kiH
        plsc.store_scatter(srcKL, [jax.lax.shift_right_logical(posL, 7), posL & 127],
                           sidx, mask=keepL)
        plsc.store_scatter(dstKL, [jax.lax.shift_right_logical(posL, 7), posL & 127],
                           didx, mask=keepL)
        plsc.store_scatter(srcKH, [jax.lax.shift_right_logical(posH, 7), posH & 127],
                           sidx, mask=keepH)
        plsc.store_scatter(dstKH, [jax.lax.shift_right_logical(posH, 7), posH & 127],
                           didx - SPLIT, mask=keepH)
        return (cntL + jnp.sum(kiL), cntH + jnp.sum(kiH))

    cntL, cntH = lax.fori_loop(0, EPW // 16, compact, (jnp.int32(0), jnp.int32(0)))

    for half, (srcK, dstK, cnt) in enumerate(((srcKL, dstKL, cntL), (srcKH, dstKH, cntH))):
        for r in range(RPS // ZCH):
            pltpu.sync_copy(rowbuf.at[pl.ds(0, ZCH)],
                            accum.at[pl.ds(s * RPS + r * ZCH, ZCH)])
        plsc.subcore_barrier()

        def chunk(j, carry, srcK=srcK, dstK=dstK):
            pltpu.async_copy(p_hbm.at[srcK.at[j]], rowbuf, gsem).wait()
            pltpu.async_copy(rowbuf, accum.at[dstK.at[j]], ssem, add=True).wait()
            return carry

        lax.fori_loop(0, (cnt + 127) // 128, chunk, 0)
        plsc.subcore_barrier()
        pltpu.sync_copy(accum.at[pl.ds(s * RPS, RPS)],
                        out_hbm.at[half, c, pl.ds(s * RPS, RPS)])
        if half == 0:
            lax.fori_loop(0, (128 * H) // 16, zrow, 0)
            plsc.subcore_barrier()


@jax.jit
def _spmm(src_pad, dst_pad, sel_pad, p_pad):
    f = pl.kernel(
        _spmm_body,
        out_type=jax.ShapeDtypeStruct((2, 2, NR, H), jnp.float32),
        mesh=plsc.VectorSubcoreMesh(core_axis_name="c", subcore_axis_name="s"),
        compiler_params=pltpu.CompilerParams(needs_layout_passes=False),
        scratch_types=[
            pltpu.VMEM((EPW,), jnp.int32),
            pltpu.VMEM((EPW,), jnp.int32),
            pltpu.VMEM((NP,), jnp.float32),
            pltpu.VMEM((NCH, 128), jnp.int32),
            pltpu.VMEM((NCH, 128), jnp.int32),
            pltpu.VMEM((NCH, 128), jnp.int32),
            pltpu.VMEM((NCH, 128), jnp.int32),
            pltpu.VMEM((128, H), jnp.float32),
            pltpu.VMEM_SHARED((NR, H), jnp.float32),
            pltpu.SemaphoreType.DMA,
            pltpu.SemaphoreType.DMA,
        ],
    )
    return f(src_pad, dst_pad, sel_pad, p_pad)


def _radix_body(stage2, bitss, inits, batch_ref, *args):
    limb_refs = args[:len(bitss)]
    out_ref = args[len(bitss)]
    batchv = batch_ref[...]
    gid = jax.lax.broadcasted_iota(jnp.int32, (B, 1), 0)
    own = gid == batchv
    counts = jnp.sum(own.astype(jnp.float32), axis=1, keepdims=True)
    kk = jnp.ceil(0.25 * counts)
    if stage2:
        kk = jnp.ceil(0.25 * kk)
    gt = jnp.zeros((B, NP), jnp.bool_)
    eq = own
    for l, (bits, init) in enumerate(zip(bitss, inits)):
        limb = limb_refs[l][...]
        cntgt = jnp.sum(gt.astype(jnp.float32), axis=1, keepdims=True)

        def round_fn(i, prefix, limb=limb, eq=eq, cntgt=cntgt, bits=bits):
            shift = jax.lax.shift_left(jnp.int32(1), jnp.int32(bits - 1) - i)
            cand = prefix + shift
            m = eq & (limb >= cand)
            cnt = cntgt + jnp.sum(m.astype(jnp.float32), axis=1, keepdims=True)
            return jnp.where(cnt >= kk, cand, prefix)

        prefix0 = jnp.full((B, 1), init, jnp.int32)
        prefix = jax.lax.fori_loop(0, bits, round_fn, prefix0)
        gt = gt | (eq & (limb > prefix))
        eq = eq & (limb == prefix)
    sel = gt | eq
    out_ref[...] = jnp.sum(sel.astype(jnp.float32), axis=0, keepdims=True)


def _topk_mask(limbs, batch_pad, stage2):
    bitss = [32] * (len(limbs) - 1) + [14]
    inits = [INTMIN] * (len(limbs) - 1) + [ID_INIT]
    limbs2d = [jnp.pad(l, (0, NP - N)).reshape(1, NP) for l in limbs]
    out = pl.pallas_call(
        functools.partial(_radix_body, stage2, bitss, inits),
        out_shape=jax.ShapeDtypeStruct((1, NP), jnp.float32),
    )(batch_pad.reshape(1, NP), *limbs2d)
    return out[0, :N] > 0.5


def _lin1_body(a1_ref, dinv0_ref, w1_ref, b1_ref, ws1_ref, h_ref, v_ref):
    h = jnp.maximum(a1_ref[...] * w1_ref[...] + b1_ref[...], 0.0)
    h_ref[...] = h
    s = jnp.dot(h, ws1_ref[...], preferred_element_type=jnp.float32)
    v_ref[...] = dinv0_ref[...] * s


def _lin1(a1_col, dinv0_col, W1, b1, Ws1):
    return pl.pallas_call(
        _lin1_body,
        out_shape=(jax.ShapeDtypeStruct((NP, H), jnp.float32),
                   jax.ShapeDtypeStruct((NP, 1), jnp.float32)),
    )(a1_col, dinv0_col, W1[0].reshape(1, H), b1.reshape(1, H), Ws1)


def _lin2_body(msum_ref, hp_ref, dinv2_ref, selc_ref, w2_ref, b2_ref, ws2_ref,
               h2_ref, v2_ref):
    dinv2 = dinv2_ref[...]
    m_total = dinv2 * msum_ref[...] + (dinv2 * dinv2) * hp_ref[...]
    h2 = selc_ref[...] * jnp.maximum(
        jnp.dot(m_total, w2_ref[...], preferred_element_type=jnp.float32) + b2_ref[...], 0.0)
    h2_ref[...] = h2
    s2 = jnp.dot(h2, ws2_ref[...], preferred_element_type=jnp.float32)
    v2_ref[...] = dinv2 * s2


def _lin2(msum_pad, hp, dinv2_col, sel_col, W2, b2, Ws2):
    return pl.pallas_call(
        _lin2_body,
        out_shape=(jax.ShapeDtypeStruct((NP, H), jnp.float32),
                   jax.ShapeDtypeStruct((NP, 1), jnp.float32)),
    )(msum_pad, hp, dinv2_col, sel_col, W2, b2.reshape(1, H), Ws2)


def _pool_core(batch_col, hval, selb):
    gid_row = jax.lax.broadcasted_iota(jnp.int32, (1, B), 1)
    ind = (batch_col == gid_row).astype(jnp.float32)
    sums = jax.lax.dot_general(ind, hval, (((0,), (0,)), ((), ())),
                               preferred_element_type=jnp.float32)
    ones_col = jnp.ones((NP, 1), jnp.float32)
    counts = jax.lax.dot_general(ind, ones_col, (((0,), (0,)), ((), ())),
                                 preferred_element_type=jnp.float32)
    neginf = jnp.float32(-jnp.inf)
    hmask = jnp.where(selb, hval, neginf)
    return ind, sums, counts, hmask


def _seg_max(batch_col, hmask):
    grow = jax.lax.broadcasted_iota(jnp.int32, (B, 1), 0)

    def gmax(g, macc):
        own = batch_col == g
        m = jnp.max(jnp.where(own, hmask, jnp.float32(-jnp.inf)), axis=0, keepdims=True)
        return jnp.where(grow == g, m, macc)

    return lax.fori_loop(0, B, gmax, jnp.zeros((B, H), jnp.float32))


def _pool1_body(batch_ref, selc_ref, hp_ref, x1_ref):
    selb = selc_ref[...] > 0.5
    hp = hp_ref[...]
    ind, sums, counts, hmask = _pool_core(batch_ref[...], hp, selb)
    k1 = jnp.ceil(0.25 * counts)
    x1_ref[:, H:] = sums / k1
    batch_col = batch_ref[...]
    x1_ref[:, :H] = _seg_max(batch_col, hmask)


def _pool1(batch_col, sel_col, hp):
    return pl.pallas_call(
        _pool1_body,
        out_shape=jax.ShapeDtypeStruct((B, 2 * H), jnp.float32),
    )(batch_col, sel_col, hp)


def _pool2_head_body(batch_ref, selc_ref, hf_ref, x1_ref, gin_ref,
                     l1w_ref, l1b_ref, l2w_ref, l2b_ref, l3w_ref, l3b_ref, out_ref):
    selb = selc_ref[...] > 0.5
    hf = hf_ref[...]
    batch_col = batch_ref[...]
    ind, sums, counts, hmask = _pool_core(batch_col, hf, selb)
    k2 = jnp.ceil(0.25 * jnp.ceil(0.25 * counts))
    x2_sum = sums / k2
    x2_max = _seg_max(batch_col, hmask)
    x1 = x1_ref[...]
    xg = jnp.concatenate([x1[:, :H] + x2_max, x1[:, H:] + x2_sum, gin_ref[...]], axis=1)
    a = jnp.maximum(jnp.dot(xg, l1w_ref[...], preferred_element_type=jnp.float32)
                    + l1b_ref[...], 0.0)
    a = jnp.maximum(jnp.dot(a, l2w_ref[...], preferred_element_type=jnp.float32)
                    + l2b_ref[...], 0.0)
    z = jnp.dot(a, l3w_ref[...], preferred_element_type=jnp.float32) + l3b_ref[...]
    zmax = jnp.max(z, axis=1, keepdims=True)
    ez = jnp.exp(z - zmax)
    lse = jnp.log(jnp.sum(ez, axis=1, keepdims=True)) + zmax
    out_ref[...] = z - lse


def _pool2_head(batch_col, sel2_col, hfin, x1, ginfo, L1w, L1b, L2w, L2b, L3w, L3b):
    FP = 384
    gin_pad = jnp.pad(ginfo, ((0, 0), (0, 128 - ginfo.shape[1])))
    l1wp = jnp.pad(L1w, ((0, FP - L1w.shape[0]), (0, 0)))
    return pl.pallas_call(
        _pool2_head_body,
        out_shape=jax.ShapeDtypeStruct((B, 32), jnp.float32),
    )(batch_col, sel2_col, hfin, x1, gin_pad, l1wp, L1b.reshape(1, H),
      L2w, L2b.reshape(1, H // 2), L3w, L3b.reshape(1, 32))


def kernel(x, edge_index, batch, ginfo, W1, b1, Ws1, bs1, W2, b2, Ws2, bs2,
           L1w, L1b, L2w, L2b, L3w, L3b):
    src, dst = edge_index[0], edge_index[1]
    padE = jnp.full((EP - E,), N, jnp.int32)
    src_pad = jnp.concatenate([src, padE])
    dst_pad = jnp.concatenate([dst, padE])

    def padn(a):
        return jnp.pad(a, (0, NP - N))

    x0 = x[:, 0]
    indeg = jnp.sum(_spmv(src_pad, dst_pad, padn(jnp.ones((N,), jnp.float32))), axis=0)[:N]
    dinv0 = (indeg + 1.0) ** -0.5
    t = jnp.sum(_spmv(src_pad, dst_pad, padn(dinv0 * x0)), axis=0)[:N]
    a1 = dinv0 * (t + dinv0 * x0)
    h, v_col = _lin1(padn(a1).reshape(NP, 1), padn(dinv0).reshape(NP, 1), W1, b1, Ws1)
    v = v_col[:N, 0]
    tv = jnp.sum(_spmv(src_pad, dst_pad, v_col[:, 0]), axis=0)[:N]
    score1 = dinv0 * (tv + v) + bs1[0]

    nid = jnp.arange(N, dtype=jnp.int32)
    batch_pad = jnp.pad(batch, (0, NP - N), constant_values=B)
    batch_col = batch_pad.reshape(NP, 1)
    key_s1 = _sortable_f32(score1)
    key_id = (~nid) ^ jnp.int32(INTMIN)
    sel1 = _topk_mask([key_s1, key_id], batch_pad, False)
    sel1f = sel1.astype(jnp.float32)
    sel_pad = padn(sel1f)
    sel_col = sel_pad.reshape(NP, 1)

    hp = padn(sel1f * jnp.tanh(score1)).reshape(NP, 1) * h
    x1 = _pool1(batch_col, sel_col, hp)

    w = jnp.sum(_spmv(src_pad, dst_pad, sel_pad), axis=0)[:N]
    dinv2 = (1.0 + sel1f * w) ** -0.5
    dinv2_col = padn(dinv2).reshape(NP, 1)
    p_pad = dinv2_col * hp
    mparts = _spmm(src_pad, dst_pad, sel_pad, p_pad)
    msum_pad = jnp.pad(
        jnp.concatenate([jnp.sum(mparts[0], axis=0)[:SPLIT],
                         jnp.sum(mparts[1], axis=0)[:N - SPLIT]], axis=0),
        ((0, NP - N), (0, 0)))
    h2p, v2_col = _lin2(msum_pad, hp, dinv2_col, sel_col, W2, b2, Ws2)
    v2 = v2_col[:N, 0]
    tv2 = jnp.sum(_spmv(src_pad, dst_pad, v2_col[:, 0]), axis=0)[:N]
    score2 = dinv2 * tv2 + dinv2 * v2 + bs2[0]

    key_s2 = _sortable_f32(score2)
    imin = jnp.int32(INTMIN)
    sel2 = _topk_mask([jnp.where(sel1, key_s2, imin), jnp.where(sel1, key_s1, imin),
                       jnp.where(sel1, key_id, imin)], batch_pad, True)
    sel2f = sel2.astype(jnp.float32)
    sel2_col = padn(sel2f).reshape(NP, 1)

    hfin = padn(sel2f * jnp.tanh(score2)).reshape(NP, 1) * h2p
    return _pool2_head(batch_col, sel2_col, hfin, x1, ginfo, L1w, L1b, L2w, L2b, L3w, L3b)
